# baseline (speedup 1.0000x reference)
import functools

import jax
import jax.numpy as jnp
from jax import lax
from jax.experimental import pallas as pl
from jax.experimental.pallas import tpu as pltpu
from jax.experimental.pallas import tpu_sc as plsc

N_NODES = 10000
NP = 10240
D = 128
E_EDGES = 320000
NCORE = 2
NSUB = 16
NW = NCORE * NSUB
EPW = E_EDGES // NW
B_EDGE = 128
NB = 80
E_PAD = NW * NB * B_EDGE
EPW_PAD = NB * B_EDGE
RPW = NP // NSUB
ACC_R = 10112
ARPW = ACC_R // NSUB
ZR = 128
ROWS_TC = 512
GRID_TC = NP // ROWS_TC

_mesh = plsc.VectorSubcoreMesh(core_axis_name="c", subcore_axis_name="s")


def _fill_batch_row_ids(rowids_ref, base):
    for i in range(NB // 16):
        rowids_ref[pl.ds(i * 16, 16)] = lax.iota(jnp.int32, 16) + (i * 16) + base


@functools.partial(
    pl.kernel,
    out_type=jax.ShapeDtypeStruct((NCORE, 2, NP), jnp.float32),
    mesh=_mesh,
    scratch_types=[
        pltpu.VMEM((NB,), jnp.int32),
        pltpu.VMEM((NB, B_EDGE), jnp.int32),
        pltpu.VMEM((NB, B_EDGE), jnp.int32),
        pltpu.VMEM((B_EDGE,), jnp.float32),
        pltpu.VMEM_SHARED((NP,), jnp.float32),
        pltpu.VMEM_SHARED((NP,), jnp.float32),
        pltpu.SemaphoreType.DMA,
        pltpu.SemaphoreType.DMA,
    ],
)
def _deg_kernel(src_hbm, dst_hbm, ones_hbm, z1_hbm, out_hbm, rowids, src_all,
                dst_all, ones_v, hsrc, hdst, sem0, sem1):
    c = lax.axis_index("c")
    s = lax.axis_index("s")
    wid = c * NSUB + s
    _fill_batch_row_ids(rowids, wid * NB)
    a0 = pltpu.async_copy(src_hbm.at[rowids], src_all, sem0)
    a1 = pltpu.async_copy(dst_hbm.at[rowids], dst_all, sem1)
    pltpu.sync_copy(z1_hbm, hsrc.at[pl.ds(s * RPW, RPW)])
    pltpu.sync_copy(z1_hbm, hdst.at[pl.ds(s * RPW, RPW)])
    pltpu.sync_copy(ones_hbm, ones_v)
    a0.wait()
    a1.wait()
    plsc.subcore_barrier()

    def body(j, carry):
        a2 = pltpu.async_copy(ones_v, hsrc.at[src_all.at[j]], sem0, add=True)
        a3 = pltpu.async_copy(ones_v, hdst.at[dst_all.at[j]], sem1, add=True)
        a2.wait()
        a3.wait()
        return carry

    lax.fori_loop(0, NB, body, 0)
    plsc.subcore_barrier()
    pltpu.sync_copy(hsrc.at[pl.ds(s * RPW, RPW)], out_hbm.at[c, 0, pl.ds(s * RPW, RPW)])
    pltpu.sync_copy(hdst.at[pl.ds(s * RPW, RPW)], out_hbm.at[c, 1, pl.ds(s * RPW, RPW)])


@functools.partial(
    pl.kernel,
    out_type=jax.ShapeDtypeStruct((NCORE, ACC_R, D), jnp.float32),
    mesh=_mesh,
    scratch_types=[
        pltpu.VMEM((NB,), jnp.int32),
        pltpu.VMEM((NB, B_EDGE), jnp.int32),
        pltpu.VMEM((B_EDGE,), jnp.int32),
        pltpu.VMEM((B_EDGE,), jnp.int32),
        pltpu.VMEM((B_EDGE, D), jnp.float32),
        pltpu.VMEM((B_EDGE, D), jnp.float32),
        pltpu.VMEM_SHARED((ACC_R, D), jnp.float32),
        pltpu.SemaphoreType.DMA,
        pltpu.SemaphoreType.DMA,
        pltpu.SemaphoreType.DMA,
        pltpu.SemaphoreType.DMA,
        pltpu.SemaphoreType.DMA,
        pltpu.SemaphoreType.DMA,
        pltpu.SemaphoreType.DMA,
    ],
)
def _agg_kernel(y_hbm, src_hbm, dst_hbm, z2_hbm, out_hbm, rowids, src_all,
                dst_v0, dst_v1, rows0, rows1, acc, semi, semd0, semd1,
                sem0, sem1, sema0, sema1):
    c = lax.axis_index("c")
    s = lax.axis_index("s")
    wid = c * NSUB + s
    rbase = wid * NB
    _fill_batch_row_ids(rowids, rbase)
    a0 = pltpu.async_copy(src_hbm.at[rowids], src_all, semi)
    pltpu.async_copy(dst_hbm.at[rbase], dst_v0, semd0)
    pltpu.async_copy(dst_hbm.at[rbase + 1], dst_v1, semd1)
    for k in range(ARPW // ZR):
        pltpu.sync_copy(z2_hbm, acc.at[pl.ds(s * ARPW + k * ZR, ZR)])
    pltpu.sync_copy(z2_hbm.at[pl.ds(0, ARPW % ZR)],
                    acc.at[pl.ds(s * ARPW + (ARPW // ZR) * ZR, ARPW % ZR)])
    a0.wait()
    pltpu.async_copy(y_hbm.at[src_all.at[0]], rows0, sem0)
    pltpu.async_copy(y_hbm.at[src_all.at[1]], rows1, sem1)
    plsc.subcore_barrier()

    def body(jj, carry):
        j = jj * 2
        jn0 = jnp.minimum(j + 2, NB - 1)
        jn1 = jnp.minimum(j + 3, NB - 1)
        pltpu.make_async_copy(y_hbm.at[src_all.at[j]], rows0, sem0).wait()
        pltpu.make_async_copy(dst_hbm.at[rbase], dst_v0, semd0).wait()
        pltpu.async_copy(rows0, acc.at[dst_v0], sema0, add=True)
        pltpu.make_async_copy(y_hbm.at[src_all.at[j + 1]], rows1, sem1).wait()
        pltpu.make_async_copy(dst_hbm.at[rbase], dst_v1, semd1).wait()
        pltpu.async_copy(rows1, acc.at[dst_v1], sema1, add=True)
        pltpu.make_async_copy(rows0, acc.at[dst_v0], sema0).wait()
        pltpu.async_copy(dst_hbm.at[rbase + jn0], dst_v0, semd0)
        pltpu.async_copy(y_hbm.at[src_all.at[jn0]], rows0, sem0)
        pltpu.make_async_copy(rows1, acc.at[dst_v1], sema1).wait()
        pltpu.async_copy(dst_hbm.at[rbase + jn1], dst_v1, semd1)
        pltpu.async_copy(y_hbm.at[src_all.at[jn1]], rows1, sem1)
        return carry

    lax.fori_loop(0, NB // 2, body, 0)
    pltpu.make_async_copy(y_hbm.at[src_all.at[NB - 1]], rows0, sem0).wait()
    pltpu.make_async_copy(y_hbm.at[src_all.at[NB - 1]], rows1, sem1).wait()
    pltpu.make_async_copy(dst_hbm.at[rbase], dst_v0, semd0).wait()
    pltpu.make_async_copy(dst_hbm.at[rbase], dst_v1, semd1).wait()
    plsc.subcore_barrier()
    pltpu.sync_copy(acc.at[pl.ds(s * ARPW, ARPW)], out_hbm.at[c, pl.ds(s * ARPW, ARPW)])


def _mm_scale_body(x_ref, w_ref, dsrc_ref, y_ref):
    deg = dsrc_ref[0, :] + dsrc_ref[1, :]
    ns = lax.rsqrt(jnp.maximum(deg, 1.0))
    y_ref[...] = jnp.dot(x_ref[...], w_ref[...],
                         preferred_element_type=jnp.float32) * ns[:, None]


_mm_scale = pl.pallas_call(
    _mm_scale_body,
    grid=(GRID_TC,),
    in_specs=[
        pl.BlockSpec((ROWS_TC, D), lambda i: (i, 0)),
        pl.BlockSpec((D, D), lambda i: (0, 0)),
        pl.BlockSpec((NCORE, ROWS_TC), lambda i: (0, i)),
    ],
    out_specs=pl.BlockSpec((ROWS_TC, D), lambda i: (i, 0)),
    out_shape=jax.ShapeDtypeStruct((NP, D), jnp.float32),
)


def _comb_mm_body(p_ref, ddst_ref, b_ref, dsrc_ref, w_ref, y_ref):
    nd = lax.rsqrt(jnp.maximum(ddst_ref[0, :] + ddst_ref[1, :], 1.0))
    h = jnp.maximum((p_ref[0] + p_ref[1]) * nd[:, None] + b_ref[...], 0.0)
    ns = lax.rsqrt(jnp.maximum(dsrc_ref[0, :] + dsrc_ref[1, :], 1.0))
    y_ref[...] = jnp.dot(h, w_ref[...],
                         preferred_element_type=jnp.float32) * ns[:, None]


_comb_mm = pl.pallas_call(
    _comb_mm_body,
    grid=(GRID_TC,),
    in_specs=[
        pl.BlockSpec((NCORE, ROWS_TC, D), lambda i: (0, i, 0)),
        pl.BlockSpec((NCORE, ROWS_TC), lambda i: (0, i)),
        pl.BlockSpec((1, D), lambda i: (0, 0)),
        pl.BlockSpec((NCORE, ROWS_TC), lambda i: (0, i)),
        pl.BlockSpec((D, D), lambda i: (0, 0)),
    ],
    out_specs=pl.BlockSpec((ROWS_TC, D), lambda i: (i, 0)),
    out_shape=jax.ShapeDtypeStruct((NP, D), jnp.float32),
)


def _final_body(p_ref, ddst_ref, b_ref, out_ref):
    nd = lax.rsqrt(jnp.maximum(ddst_ref[0, :] + ddst_ref[1, :], 1.0))
    z = jnp.maximum((p_ref[0] + p_ref[1]) * nd[:, None] + b_ref[...], 0.0)
    z = z - jnp.max(z, axis=1, keepdims=True)
    e = jnp.exp(z)
    out_ref[...] = e / jnp.sum(e, axis=1, keepdims=True)


_final = pl.pallas_call(
    _final_body,
    grid=(GRID_TC,),
    in_specs=[
        pl.BlockSpec((NCORE, ROWS_TC, D), lambda i: (0, i, 0)),
        pl.BlockSpec((NCORE, ROWS_TC), lambda i: (0, i)),
        pl.BlockSpec((1, D), lambda i: (0, 0)),
    ],
    out_specs=pl.BlockSpec((ROWS_TC, D), lambda i: (i, 0)),
    out_shape=jax.ShapeDtypeStruct((N_NODES, D), jnp.float32),
)


def _mm_body(x_ref, w_ref, y_ref):
    y_ref[...] = jnp.dot(x_ref[...], w_ref[...],
                         preferred_element_type=jnp.float32)


_mm = pl.pallas_call(
    _mm_body,
    grid=(GRID_TC,),
    in_specs=[
        pl.BlockSpec((ROWS_TC, D), lambda i: (i, 0)),
        pl.BlockSpec((D, D), lambda i: (0, 0)),
    ],
    out_specs=pl.BlockSpec((ROWS_TC, D), lambda i: (i, 0)),
    out_shape=jax.ShapeDtypeStruct((NP, D), jnp.float32),
)


def _scale_body(x_ref, dsrc_ref, y_ref):
    ns = lax.rsqrt(jnp.maximum(dsrc_ref[0, :] + dsrc_ref[1, :], 1.0))
    y_ref[...] = x_ref[...] * ns[:, None]


_scale = pl.pallas_call(
    _scale_body,
    grid=(GRID_TC,),
    in_specs=[
        pl.BlockSpec((ROWS_TC, D), lambda i: (i, 0)),
        pl.BlockSpec((NCORE, ROWS_TC), lambda i: (0, i)),
    ],
    out_specs=pl.BlockSpec((ROWS_TC, D), lambda i: (i, 0)),
    out_shape=jax.ShapeDtypeStruct((NP, D), jnp.float32),
)


def kernel(in_feat, edge_index, W1, b1, W2, b2):
    npad = E_PAD - E_EDGES
    pad = N_NODES + (jnp.arange(npad, dtype=jnp.int32) % (ACC_R - N_NODES))
    src2 = jnp.concatenate([edge_index[0], pad]).reshape(NW * NB, B_EDGE)
    dst2 = jnp.concatenate([edge_index[1], pad]).reshape(NW * NB, B_EDGE)
    ones = jnp.ones((B_EDGE,), jnp.float32)
    z1 = jnp.zeros((RPW,), jnp.float32)
    z2 = jnp.zeros((ZR, D), jnp.float32)
    degp = _deg_kernel(src2, dst2, ones, z1)
    dsrc = degp[:, 0, :]
    ddst = degp[:, 1, :]
    b1r = b1.reshape(1, D)
    b2r = b2.reshape(1, D)
    y1 = _mm_scale(in_feat, W1, dsrc)
    p1 = _agg_kernel(y1, src2, dst2, z2)
    y2 = _comb_mm(p1, ddst, b1r, dsrc, W2)
    p2 = _agg_kernel(y2, src2, dst2, z2)
    return _final(p2, ddst, b2r)


# speedup vs baseline: 1.2161x; 1.2161x over previous
import functools

import jax
import jax.numpy as jnp
from jax import lax
from jax.experimental import pallas as pl
from jax.experimental.pallas import tpu as pltpu
from jax.experimental.pallas import tpu_sc as plsc

N_NODES = 10000
NP = 10240
D = 128
E_EDGES = 320000
NCORE = 2
NSUB = 16
NW = NCORE * NSUB
EPW = E_EDGES // NW
B_EDGE = 128
NBT = E_EDGES // B_EDGE
NB_FLOOR = NBT // NW
REM = NBT - NB_FLOOR * NW
NB_MAX = NB_FLOOR + 1
RPW = NP // NSUB
ACC_R = 10112
ARPW = ACC_R // NSUB
ZR = 128
ROWS_TC = 512
GRID_TC = NP // ROWS_TC

_mesh = plsc.VectorSubcoreMesh(core_axis_name="c", subcore_axis_name="s")


def _fill_batch_row_ids(rowids_ref, base, nb):
    for i in range((NB_MAX + 15) // 16 * 16 // 16):
        loc = jnp.minimum(lax.iota(jnp.int32, 16) + (i * 16), nb - 1)
        rowids_ref[pl.ds(i * 16, 16)] = loc + base


@functools.partial(
    pl.kernel,
    out_type=jax.ShapeDtypeStruct((NCORE, 2, NP), jnp.float32),
    mesh=_mesh,
    scratch_types=[
        pltpu.VMEM((80,), jnp.int32),
        pltpu.VMEM((80, B_EDGE), jnp.int32),
        pltpu.VMEM((80, B_EDGE), jnp.int32),
        pltpu.VMEM((B_EDGE,), jnp.float32),
        pltpu.VMEM_SHARED((NP,), jnp.float32),
        pltpu.VMEM_SHARED((NP,), jnp.float32),
        pltpu.SemaphoreType.DMA,
        pltpu.SemaphoreType.DMA,
    ],
)
def _deg_kernel(src_hbm, dst_hbm, ones_hbm, z1_hbm, out_hbm, rowids, src_all,
                dst_all, ones_v, hsrc, hdst, sem0, sem1):
    c = lax.axis_index("c")
    s = lax.axis_index("s")
    wid = c * NSUB + s
    nb = NB_FLOOR + jnp.where(wid < REM, 1, 0)
    rbase = wid * NB_FLOOR + jnp.minimum(wid, REM)
    _fill_batch_row_ids(rowids, rbase, nb)
    a0 = pltpu.async_copy(src_hbm.at[rowids], src_all, sem0)
    a1 = pltpu.async_copy(dst_hbm.at[rowids], dst_all, sem1)
    pltpu.sync_copy(z1_hbm, hsrc.at[pl.ds(s * RPW, RPW)])
    pltpu.sync_copy(z1_hbm, hdst.at[pl.ds(s * RPW, RPW)])
    pltpu.sync_copy(ones_hbm, ones_v)
    a0.wait()
    a1.wait()
    plsc.subcore_barrier()

    def body(j, carry):
        a2 = pltpu.async_copy(ones_v, hsrc.at[src_all.at[j]], sem0, add=True)
        a3 = pltpu.async_copy(ones_v, hdst.at[dst_all.at[j]], sem1, add=True)
        a2.wait()
        a3.wait()
        return carry

    lax.fori_loop(0, nb, body, 0)
    plsc.subcore_barrier()
    pltpu.sync_copy(hsrc.at[pl.ds(s * RPW, RPW)], out_hbm.at[c, 0, pl.ds(s * RPW, RPW)])
    pltpu.sync_copy(hdst.at[pl.ds(s * RPW, RPW)], out_hbm.at[c, 1, pl.ds(s * RPW, RPW)])


@functools.partial(
    pl.kernel,
    out_type=jax.ShapeDtypeStruct((NCORE, ACC_R, D), jnp.float32),
    mesh=_mesh,
    scratch_types=[
        pltpu.VMEM((80,), jnp.int32),
        pltpu.VMEM((80, B_EDGE), jnp.int32),
        pltpu.VMEM((B_EDGE,), jnp.int32),
        pltpu.VMEM((B_EDGE,), jnp.int32),
        pltpu.VMEM((B_EDGE, D), jnp.float32),
        pltpu.VMEM((B_EDGE, D), jnp.float32),
        pltpu.VMEM_SHARED((ACC_R, D), jnp.float32),
        pltpu.SemaphoreType.DMA,
        pltpu.SemaphoreType.DMA,
        pltpu.SemaphoreType.DMA,
        pltpu.SemaphoreType.DMA,
        pltpu.SemaphoreType.DMA,
    ],
)
def _agg_kernel(y_hbm, src_hbm, dst_hbm, z2_hbm, out_hbm, rowids, src_all,
                dst_v0, dst_v1, rows0, rows1, acc, semi, semd0, semd1,
                sem0, sem1):
    c = lax.axis_index("c")
    s = lax.axis_index("s")
    wid = c * NSUB + s
    nb = NB_FLOOR + jnp.where(wid < REM, 1, 0)
    rbase = wid * NB_FLOOR + jnp.minimum(wid, REM)
    _fill_batch_row_ids(rowids, rbase, nb)
    a0 = pltpu.async_copy(src_hbm.at[rowids], src_all, semi)
    pltpu.async_copy(dst_hbm.at[rbase], dst_v0, semd0)
    pltpu.async_copy(dst_hbm.at[rbase + 1], dst_v1, semd1)
    for k in range(ARPW // ZR):
        pltpu.sync_copy(z2_hbm, acc.at[pl.ds(s * ARPW + k * ZR, ZR)])
    pltpu.sync_copy(z2_hbm.at[pl.ds(0, ARPW % ZR)],
                    acc.at[pl.ds(s * ARPW + (ARPW // ZR) * ZR, ARPW % ZR)])
    a0.wait()
    pltpu.async_copy(y_hbm.at[src_all.at[0]], rows0, sem0)
    pltpu.async_copy(y_hbm.at[src_all.at[1]], rows1, sem1)
    plsc.subcore_barrier()

    def body(jj, carry):
        j = jj * 2
        jn0 = jnp.minimum(j + 2, nb - 1)
        jn1 = jnp.minimum(j + 3, nb - 1)
        pltpu.make_async_copy(y_hbm.at[src_all.at[j]], rows0, sem0).wait()
        pltpu.make_async_copy(dst_hbm.at[rbase], dst_v0, semd0).wait()
        pltpu.sync_copy(rows0, acc.at[dst_v0], add=True)
        pltpu.async_copy(dst_hbm.at[rbase + jn0], dst_v0, semd0)
        pltpu.async_copy(y_hbm.at[src_all.at[jn0]], rows0, sem0)
        pltpu.make_async_copy(y_hbm.at[src_all.at[j + 1]], rows1, sem1).wait()
        pltpu.make_async_copy(dst_hbm.at[rbase], dst_v1, semd1).wait()
        pltpu.sync_copy(rows1, acc.at[dst_v1], add=True)
        pltpu.async_copy(dst_hbm.at[rbase + jn1], dst_v1, semd1)
        pltpu.async_copy(y_hbm.at[src_all.at[jn1]], rows1, sem1)
        return carry

    lax.fori_loop(0, NB_FLOOR // 2, body, 0)
    pltpu.make_async_copy(y_hbm.at[src_all.at[0]], rows0, sem0).wait()
    pltpu.make_async_copy(dst_hbm.at[rbase], dst_v0, semd0).wait()

    @pl.when(wid < REM)
    def _():
        pltpu.sync_copy(rows0, acc.at[dst_v0], add=True)

    pltpu.make_async_copy(y_hbm.at[src_all.at[0]], rows1, sem1).wait()
    pltpu.make_async_copy(dst_hbm.at[rbase], dst_v1, semd1).wait()
    plsc.subcore_barrier()
    pltpu.sync_copy(acc.at[pl.ds(s * ARPW, ARPW)], out_hbm.at[c, pl.ds(s * ARPW, ARPW)])


def _mm_scale_body(x_ref, w_ref, dsrc_ref, y_ref):
    deg = dsrc_ref[0, :] + dsrc_ref[1, :]
    ns = lax.rsqrt(jnp.maximum(deg, 1.0))
    y_ref[...] = jnp.dot(x_ref[...], w_ref[...],
                         preferred_element_type=jnp.float32) * ns[:, None]


_mm_scale = pl.pallas_call(
    _mm_scale_body,
    grid=(GRID_TC,),
    in_specs=[
        pl.BlockSpec((ROWS_TC, D), lambda i: (i, 0)),
        pl.BlockSpec((D, D), lambda i: (0, 0)),
        pl.BlockSpec((NCORE, ROWS_TC), lambda i: (0, i)),
    ],
    out_specs=pl.BlockSpec((ROWS_TC, D), lambda i: (i, 0)),
    out_shape=jax.ShapeDtypeStruct((NP, D), jnp.float32),
)


def _comb_mm_body(p_ref, ddst_ref, b_ref, dsrc_ref, w_ref, y_ref):
    nd = lax.rsqrt(jnp.maximum(ddst_ref[0, :] + ddst_ref[1, :], 1.0))
    h = jnp.maximum((p_ref[0] + p_ref[1]) * nd[:, None] + b_ref[...], 0.0)
    ns = lax.rsqrt(jnp.maximum(dsrc_ref[0, :] + dsrc_ref[1, :], 1.0))
    y_ref[...] = jnp.dot(h, w_ref[...],
                         preferred_element_type=jnp.float32) * ns[:, None]


_comb_mm = pl.pallas_call(
    _comb_mm_body,
    grid=(GRID_TC,),
    in_specs=[
        pl.BlockSpec((NCORE, ROWS_TC, D), lambda i: (0, i, 0)),
        pl.BlockSpec((NCORE, ROWS_TC), lambda i: (0, i)),
        pl.BlockSpec((1, D), lambda i: (0, 0)),
        pl.BlockSpec((NCORE, ROWS_TC), lambda i: (0, i)),
        pl.BlockSpec((D, D), lambda i: (0, 0)),
    ],
    out_specs=pl.BlockSpec((ROWS_TC, D), lambda i: (i, 0)),
    out_shape=jax.ShapeDtypeStruct((NP, D), jnp.float32),
)


def _final_body(p_ref, ddst_ref, b_ref, out_ref):
    nd = lax.rsqrt(jnp.maximum(ddst_ref[0, :] + ddst_ref[1, :], 1.0))
    z = jnp.maximum((p_ref[0] + p_ref[1]) * nd[:, None] + b_ref[...], 0.0)
    z = z - jnp.max(z, axis=1, keepdims=True)
    e = jnp.exp(z)
    out_ref[...] = e / jnp.sum(e, axis=1, keepdims=True)


_final = pl.pallas_call(
    _final_body,
    grid=(GRID_TC,),
    in_specs=[
        pl.BlockSpec((NCORE, ROWS_TC, D), lambda i: (0, i, 0)),
        pl.BlockSpec((NCORE, ROWS_TC), lambda i: (0, i)),
        pl.BlockSpec((1, D), lambda i: (0, 0)),
    ],
    out_specs=pl.BlockSpec((ROWS_TC, D), lambda i: (i, 0)),
    out_shape=jax.ShapeDtypeStruct((N_NODES, D), jnp.float32),
)


def kernel(in_feat, edge_index, W1, b1, W2, b2):
    src2 = edge_index[0].reshape(NBT, B_EDGE)
    dst2 = edge_index[1].reshape(NBT, B_EDGE)
    ones = jnp.ones((B_EDGE,), jnp.float32)
    z1 = jnp.zeros((RPW,), jnp.float32)
    z2 = jnp.zeros((ZR, D), jnp.float32)
    degp = _deg_kernel(src2, dst2, ones, z1)
    dsrc = degp[:, 0, :]
    ddst = degp[:, 1, :]
    b1r = b1.reshape(1, D)
    b2r = b2.reshape(1, D)
    y1 = _mm_scale(in_feat, W1, dsrc)
    p1 = _agg_kernel(y1, src2, dst2, z2)
    y2 = _comb_mm(p1, ddst, b1r, dsrc, W2)
    p2 = _agg_kernel(y2, src2, dst2, z2)
    return _final(p2, ddst, b2r)


# speedup vs baseline: 1.2886x; 1.0596x over previous
import functools

import jax
import jax.numpy as jnp
from jax import lax
from jax.experimental import pallas as pl
from jax.experimental.pallas import tpu as pltpu
from jax.experimental.pallas import tpu_sc as plsc

N_NODES = 10000
NP = 10240
D = 128
E_EDGES = 320000
NCORE = 2
NSUB = 16
NW = NCORE * NSUB
EPW = E_EDGES // NW
B_EDGE = 128
NBT = E_EDGES // B_EDGE
NB_FLOOR = NBT // NW
REM = NBT - NB_FLOOR * NW
NB_MAX = NB_FLOOR + 1
RPW = NP // NSUB
ACC_R = 10112
ARPW = ACC_R // NSUB
ZR = 128
ROWS_TC = 512
GRID_TC = NP // ROWS_TC

_mesh = plsc.VectorSubcoreMesh(core_axis_name="c", subcore_axis_name="s")


def _fill_batch_row_ids(rowids_ref, base, nb):
    for i in range((NB_MAX + 15) // 16 * 16 // 16):
        loc = jnp.minimum(lax.iota(jnp.int32, 16) + (i * 16), nb - 1)
        rowids_ref[pl.ds(i * 16, 16)] = loc + base


@functools.partial(
    pl.kernel,
    out_type=jax.ShapeDtypeStruct((NCORE, 2, NP), jnp.float32),
    mesh=_mesh,
    scratch_types=[
        pltpu.VMEM((80,), jnp.int32),
        pltpu.VMEM((80,), jnp.int32),
        pltpu.VMEM((80, B_EDGE), jnp.int32),
        pltpu.VMEM((80, B_EDGE), jnp.int32),
        pltpu.VMEM((B_EDGE,), jnp.float32),
        pltpu.VMEM_SHARED((NP,), jnp.float32),
        pltpu.VMEM_SHARED((NP,), jnp.float32),
        pltpu.SemaphoreType.DMA,
        pltpu.SemaphoreType.DMA,
    ],
)
def _deg_kernel(e2_hbm, ones_hbm, z1_hbm, out_hbm, rowids, rowids2, src_all,
                dst_all, ones_v, hsrc, hdst, sem0, sem1):
    c = lax.axis_index("c")
    s = lax.axis_index("s")
    wid = c * NSUB + s
    nb = NB_FLOOR + jnp.where(wid < REM, 1, 0)
    rbase = wid * NB_FLOOR + jnp.minimum(wid, REM)
    _fill_batch_row_ids(rowids, rbase, nb)
    _fill_batch_row_ids(rowids2, NBT + rbase, nb)
    a0 = pltpu.async_copy(e2_hbm.at[rowids], src_all, sem0)
    a1 = pltpu.async_copy(e2_hbm.at[rowids2], dst_all, sem1)
    pltpu.sync_copy(z1_hbm, hsrc.at[pl.ds(s * RPW, RPW)])
    pltpu.sync_copy(z1_hbm, hdst.at[pl.ds(s * RPW, RPW)])
    pltpu.sync_copy(ones_hbm, ones_v)
    a0.wait()
    a1.wait()
    plsc.subcore_barrier()

    def body(j, carry):
        a2 = pltpu.async_copy(ones_v, hsrc.at[src_all.at[j]], sem0, add=True)
        a3 = pltpu.async_copy(ones_v, hdst.at[dst_all.at[j]], sem1, add=True)
        a2.wait()
        a3.wait()
        return carry

    lax.fori_loop(0, nb, body, 0)
    plsc.subcore_barrier()
    pltpu.sync_copy(hsrc.at[pl.ds(s * RPW, RPW)], out_hbm.at[c, 0, pl.ds(s * RPW, RPW)])
    pltpu.sync_copy(hdst.at[pl.ds(s * RPW, RPW)], out_hbm.at[c, 1, pl.ds(s * RPW, RPW)])


@functools.partial(
    pl.kernel,
    out_type=jax.ShapeDtypeStruct((NCORE, ACC_R, D), jnp.float32),
    mesh=_mesh,
    scratch_types=[
        pltpu.VMEM((80,), jnp.int32),
        pltpu.VMEM((80, B_EDGE), jnp.int32),
        pltpu.VMEM((B_EDGE,), jnp.int32),
        pltpu.VMEM((B_EDGE,), jnp.int32),
        pltpu.VMEM((B_EDGE, D), jnp.float32),
        pltpu.VMEM((B_EDGE, D), jnp.float32),
        pltpu.VMEM_SHARED((ACC_R, D), jnp.float32),
        pltpu.SemaphoreType.DMA,
        pltpu.SemaphoreType.DMA,
        pltpu.SemaphoreType.DMA,
        pltpu.SemaphoreType.DMA,
        pltpu.SemaphoreType.DMA,
    ],
)
def _agg_kernel(y_hbm, e2_hbm, z2_hbm, out_hbm, rowids, src_all,
                dst_v0, dst_v1, rows0, rows1, acc, semi, semd0, semd1,
                sem0, sem1):
    c = lax.axis_index("c")
    s = lax.axis_index("s")
    wid = c * NSUB + s
    nb = NB_FLOOR + jnp.where(wid < REM, 1, 0)
    rbase = wid * NB_FLOOR + jnp.minimum(wid, REM)
    dbase = NBT + rbase
    _fill_batch_row_ids(rowids, rbase, nb)
    a0 = pltpu.async_copy(e2_hbm.at[rowids], src_all, semi)
    pltpu.async_copy(e2_hbm.at[dbase], dst_v0, semd0)
    pltpu.async_copy(e2_hbm.at[dbase + 1], dst_v1, semd1)
    for k in range(ARPW // ZR):
        pltpu.sync_copy(z2_hbm, acc.at[pl.ds(s * ARPW + k * ZR, ZR)])
    pltpu.sync_copy(z2_hbm.at[pl.ds(0, ARPW % ZR)],
                    acc.at[pl.ds(s * ARPW + (ARPW // ZR) * ZR, ARPW % ZR)])
    a0.wait()
    pltpu.async_copy(y_hbm.at[src_all.at[0]], rows0, sem0)
    pltpu.async_copy(y_hbm.at[src_all.at[1]], rows1, sem1)
    plsc.subcore_barrier()

    def body(jj, carry):
        j = jj * 2
        jn0 = jnp.minimum(j + 2, nb - 1)
        jn1 = jnp.minimum(j + 3, nb - 1)
        pltpu.make_async_copy(y_hbm.at[src_all.at[j]], rows0, sem0).wait()
        pltpu.make_async_copy(e2_hbm.at[dbase], dst_v0, semd0).wait()
        pltpu.sync_copy(rows0, acc.at[dst_v0], add=True)
        pltpu.async_copy(e2_hbm.at[dbase + jn0], dst_v0, semd0)
        pltpu.async_copy(y_hbm.at[src_all.at[jn0]], rows0, sem0)
        pltpu.make_async_copy(y_hbm.at[src_all.at[j + 1]], rows1, sem1).wait()
        pltpu.make_async_copy(e2_hbm.at[dbase], dst_v1, semd1).wait()
        pltpu.sync_copy(rows1, acc.at[dst_v1], add=True)
        pltpu.async_copy(e2_hbm.at[dbase + jn1], dst_v1, semd1)
        pltpu.async_copy(y_hbm.at[src_all.at[jn1]], rows1, sem1)
        return carry

    lax.fori_loop(0, NB_FLOOR // 2, body, 0)
    pltpu.make_async_copy(y_hbm.at[src_all.at[0]], rows0, sem0).wait()
    pltpu.make_async_copy(e2_hbm.at[dbase], dst_v0, semd0).wait()

    @pl.when(wid < REM)
    def _():
        pltpu.sync_copy(rows0, acc.at[dst_v0], add=True)

    pltpu.make_async_copy(y_hbm.at[src_all.at[0]], rows1, sem1).wait()
    pltpu.make_async_copy(e2_hbm.at[dbase], dst_v1, semd1).wait()
    plsc.subcore_barrier()
    pltpu.sync_copy(acc.at[pl.ds(s * ARPW, ARPW)], out_hbm.at[c, pl.ds(s * ARPW, ARPW)])


def _mm_scale_body(x_ref, w_ref, dsrc_ref, y_ref):
    deg = dsrc_ref[0, :] + dsrc_ref[1, :]
    ns = lax.rsqrt(jnp.maximum(deg, 1.0))
    y_ref[...] = jnp.dot(x_ref[...], w_ref[...],
                         preferred_element_type=jnp.float32) * ns[:, None]


_mm_scale = pl.pallas_call(
    _mm_scale_body,
    grid=(GRID_TC,),
    in_specs=[
        pl.BlockSpec((ROWS_TC, D), lambda i: (i, 0)),
        pl.BlockSpec((D, D), lambda i: (0, 0)),
        pl.BlockSpec((NCORE, ROWS_TC), lambda i: (0, i)),
    ],
    out_specs=pl.BlockSpec((ROWS_TC, D), lambda i: (i, 0)),
    out_shape=jax.ShapeDtypeStruct((NP, D), jnp.float32),
)


def _comb_mm_body(p_ref, ddst_ref, b_ref, dsrc_ref, w_ref, y_ref):
    nd = lax.rsqrt(jnp.maximum(ddst_ref[0, :] + ddst_ref[1, :], 1.0))
    h = jnp.maximum((p_ref[0] + p_ref[1]) * nd[:, None] + b_ref[...], 0.0)
    ns = lax.rsqrt(jnp.maximum(dsrc_ref[0, :] + dsrc_ref[1, :], 1.0))
    y_ref[...] = jnp.dot(h, w_ref[...],
                         preferred_element_type=jnp.float32) * ns[:, None]


_comb_mm = pl.pallas_call(
    _comb_mm_body,
    grid=(GRID_TC,),
    in_specs=[
        pl.BlockSpec((NCORE, ROWS_TC, D), lambda i: (0, i, 0)),
        pl.BlockSpec((NCORE, ROWS_TC), lambda i: (0, i)),
        pl.BlockSpec((1, D), lambda i: (0, 0)),
        pl.BlockSpec((NCORE, ROWS_TC), lambda i: (0, i)),
        pl.BlockSpec((D, D), lambda i: (0, 0)),
    ],
    out_specs=pl.BlockSpec((ROWS_TC, D), lambda i: (i, 0)),
    out_shape=jax.ShapeDtypeStruct((NP, D), jnp.float32),
)


def _final_body(p_ref, ddst_ref, b_ref, out_ref):
    nd = lax.rsqrt(jnp.maximum(ddst_ref[0, :] + ddst_ref[1, :], 1.0))
    z = jnp.maximum((p_ref[0] + p_ref[1]) * nd[:, None] + b_ref[...], 0.0)
    z = z - jnp.max(z, axis=1, keepdims=True)
    e = jnp.exp(z)
    out_ref[...] = e / jnp.sum(e, axis=1, keepdims=True)


_final = pl.pallas_call(
    _final_body,
    grid=(GRID_TC,),
    in_specs=[
        pl.BlockSpec((NCORE, ROWS_TC, D), lambda i: (0, i, 0)),
        pl.BlockSpec((NCORE, ROWS_TC), lambda i: (0, i)),
        pl.BlockSpec((1, D), lambda i: (0, 0)),
    ],
    out_specs=pl.BlockSpec((ROWS_TC, D), lambda i: (i, 0)),
    out_shape=jax.ShapeDtypeStruct((N_NODES, D), jnp.float32),
)


def kernel(in_feat, edge_index, W1, b1, W2, b2):
    e2 = edge_index.reshape(2 * NBT, B_EDGE)
    ones = jnp.ones((B_EDGE,), jnp.float32)
    z1 = jnp.zeros((RPW,), jnp.float32)
    z2 = jnp.zeros((ZR, D), jnp.float32)
    degp = _deg_kernel(e2, ones, z1)
    dsrc = degp[:, 0, :]
    ddst = degp[:, 1, :]
    b1r = b1.reshape(1, D)
    b2r = b2.reshape(1, D)
    y1 = _mm_scale(in_feat, W1, dsrc)
    p1 = _agg_kernel(y1, e2, z2)
    y2 = _comb_mm(p1, ddst, b1r, dsrc, W2)
    p2 = _agg_kernel(y2, e2, z2)
    return _final(p2, ddst, b2r)


# speedup vs baseline: 1.3559x; 1.0523x over previous
import functools

import jax
import jax.numpy as jnp
from jax import lax
from jax.experimental import pallas as pl
from jax.experimental.pallas import tpu as pltpu
from jax.experimental.pallas import tpu_sc as plsc

N_NODES = 10000
NP = 10240
D = 128
E_EDGES = 320000
NCORE = 2
NSUB = 16
NW = NCORE * NSUB
EPW = E_EDGES // NW
B_EDGE = 128
NBT = E_EDGES // B_EDGE
NB_FLOOR = NBT // NW
REM = NBT - NB_FLOOR * NW
NB_MAX = NB_FLOOR + 1
RPW = NP // NSUB
ACC_R = 10112
ARPW = ACC_R // NSUB
ZR = 128
ROWS_TC = 1024
GRID_TC = NP // ROWS_TC

_mesh = plsc.VectorSubcoreMesh(core_axis_name="c", subcore_axis_name="s")


def _fill_batch_row_ids(rowids_ref, base, nb):
    for i in range((NB_MAX + 15) // 16 * 16 // 16):
        loc = jnp.minimum(lax.iota(jnp.int32, 16) + (i * 16), nb - 1)
        rowids_ref[pl.ds(i * 16, 16)] = loc + base


@functools.partial(
    pl.kernel,
    out_type=jax.ShapeDtypeStruct((NCORE, 2, NP), jnp.float32),
    mesh=_mesh,
    scratch_types=[
        pltpu.VMEM((80,), jnp.int32),
        pltpu.VMEM((80,), jnp.int32),
        pltpu.VMEM((80, B_EDGE), jnp.int32),
        pltpu.VMEM((80, B_EDGE), jnp.int32),
        pltpu.VMEM((B_EDGE,), jnp.float32),
        pltpu.VMEM_SHARED((NP,), jnp.float32),
        pltpu.VMEM_SHARED((NP,), jnp.float32),
        pltpu.SemaphoreType.DMA,
        pltpu.SemaphoreType.DMA,
    ],
)
def _deg_kernel(e2_hbm, ones_hbm, z1_hbm, out_hbm, rowids, rowids2, src_all,
                dst_all, ones_v, hsrc, hdst, sem0, sem1):
    c = lax.axis_index("c")
    s = lax.axis_index("s")
    wid = c * NSUB + s
    nb = NB_FLOOR + jnp.where(wid < REM, 1, 0)
    rbase = wid * NB_FLOOR + jnp.minimum(wid, REM)
    _fill_batch_row_ids(rowids, rbase, nb)
    _fill_batch_row_ids(rowids2, NBT + rbase, nb)
    a0 = pltpu.async_copy(e2_hbm.at[rowids], src_all, sem0)
    a1 = pltpu.async_copy(e2_hbm.at[rowids2], dst_all, sem1)
    pltpu.sync_copy(z1_hbm, hsrc.at[pl.ds(s * RPW, RPW)])
    pltpu.sync_copy(z1_hbm, hdst.at[pl.ds(s * RPW, RPW)])
    pltpu.sync_copy(ones_hbm, ones_v)
    a0.wait()
    a1.wait()
    plsc.subcore_barrier()

    def body(j, carry):
        a2 = pltpu.async_copy(ones_v, hsrc.at[src_all.at[j]], sem0, add=True)
        a3 = pltpu.async_copy(ones_v, hdst.at[dst_all.at[j]], sem1, add=True)
        a2.wait()
        a3.wait()
        return carry

    lax.fori_loop(0, nb, body, 0)
    plsc.subcore_barrier()
    pltpu.sync_copy(hsrc.at[pl.ds(s * RPW, RPW)], out_hbm.at[c, 0, pl.ds(s * RPW, RPW)])
    pltpu.sync_copy(hdst.at[pl.ds(s * RPW, RPW)], out_hbm.at[c, 1, pl.ds(s * RPW, RPW)])


@functools.partial(
    pl.kernel,
    out_type=jax.ShapeDtypeStruct((NCORE, ACC_R, D), jnp.float32),
    mesh=_mesh,
    scratch_types=[
        pltpu.VMEM((80,), jnp.int32),
        pltpu.VMEM((80, B_EDGE), jnp.int32),
        pltpu.VMEM((B_EDGE,), jnp.int32),
        pltpu.VMEM((B_EDGE,), jnp.int32),
        pltpu.VMEM((B_EDGE, D), jnp.float32),
        pltpu.VMEM((B_EDGE, D), jnp.float32),
        pltpu.VMEM_SHARED((ACC_R, D), jnp.float32),
        pltpu.SemaphoreType.DMA,
        pltpu.SemaphoreType.DMA,
        pltpu.SemaphoreType.DMA,
        pltpu.SemaphoreType.DMA,
        pltpu.SemaphoreType.DMA,
    ],
)
def _agg_kernel(y_hbm, e2_hbm, z2_hbm, out_hbm, rowids, src_all,
                dst_v0, dst_v1, rows0, rows1, acc, semi, semd0, semd1,
                sem0, sem1):
    c = lax.axis_index("c")
    s = lax.axis_index("s")
    wid = c * NSUB + s
    nb = NB_FLOOR + jnp.where(wid < REM, 1, 0)
    rbase = wid * NB_FLOOR + jnp.minimum(wid, REM)
    dbase = NBT + rbase
    _fill_batch_row_ids(rowids, rbase, nb)
    a0 = pltpu.async_copy(e2_hbm.at[rowids], src_all, semi)
    pltpu.async_copy(e2_hbm.at[dbase], dst_v0, semd0)
    pltpu.async_copy(e2_hbm.at[dbase + 1], dst_v1, semd1)
    for k in range(ARPW // ZR):
        pltpu.sync_copy(z2_hbm, acc.at[pl.ds(s * ARPW + k * ZR, ZR)])
    pltpu.sync_copy(z2_hbm.at[pl.ds(0, ARPW % ZR)],
                    acc.at[pl.ds(s * ARPW + (ARPW // ZR) * ZR, ARPW % ZR)])
    a0.wait()
    pltpu.async_copy(y_hbm.at[src_all.at[0]], rows0, sem0)
    pltpu.async_copy(y_hbm.at[src_all.at[1]], rows1, sem1)
    plsc.subcore_barrier()

    def body(jj, carry):
        j = jj * 2
        jn0 = jnp.minimum(j + 2, nb - 1)
        jn1 = jnp.minimum(j + 3, nb - 1)
        pltpu.make_async_copy(z2_hbm, rows0, sem0).wait()
        pltpu.make_async_copy(e2_hbm.at[dbase], dst_v0, semd0).wait()
        pltpu.sync_copy(rows0, acc.at[dst_v0], add=True)
        pltpu.async_copy(e2_hbm.at[dbase + jn0], dst_v0, semd0)
        pltpu.async_copy(y_hbm.at[src_all.at[jn0]], rows0, sem0)
        pltpu.make_async_copy(z2_hbm, rows1, sem1).wait()
        pltpu.make_async_copy(e2_hbm.at[dbase], dst_v1, semd1).wait()
        pltpu.sync_copy(rows1, acc.at[dst_v1], add=True)
        pltpu.async_copy(e2_hbm.at[dbase + jn1], dst_v1, semd1)
        pltpu.async_copy(y_hbm.at[src_all.at[jn1]], rows1, sem1)
        return carry

    lax.fori_loop(0, NB_FLOOR // 2, body, 0)
    pltpu.make_async_copy(z2_hbm, rows0, sem0).wait()
    pltpu.make_async_copy(e2_hbm.at[dbase], dst_v0, semd0).wait()

    @pl.when(wid < REM)
    def _():
        pltpu.sync_copy(rows0, acc.at[dst_v0], add=True)

    pltpu.make_async_copy(z2_hbm, rows1, sem1).wait()
    pltpu.make_async_copy(e2_hbm.at[dbase], dst_v1, semd1).wait()
    plsc.subcore_barrier()
    pltpu.sync_copy(acc.at[pl.ds(s * ARPW, ARPW)], out_hbm.at[c, pl.ds(s * ARPW, ARPW)])


def _mm_scale_body(x_ref, w_ref, dsrc_ref, y_ref):
    deg = dsrc_ref[0, :] + dsrc_ref[1, :]
    ns = lax.rsqrt(jnp.maximum(deg, 1.0))
    y_ref[...] = jnp.dot(x_ref[...], w_ref[...],
                         preferred_element_type=jnp.float32) * ns[:, None]


_mm_scale = pl.pallas_call(
    _mm_scale_body,
    grid=(GRID_TC,),
    in_specs=[
        pl.BlockSpec((ROWS_TC, D), lambda i: (i, 0)),
        pl.BlockSpec((D, D), lambda i: (0, 0)),
        pl.BlockSpec((NCORE, ROWS_TC), lambda i: (0, i)),
    ],
    out_specs=pl.BlockSpec((ROWS_TC, D), lambda i: (i, 0)),
    out_shape=jax.ShapeDtypeStruct((NP, D), jnp.float32),
)


def _comb_mm_body(p_ref, ddst_ref, b_ref, dsrc_ref, w_ref, y_ref):
    nd = lax.rsqrt(jnp.maximum(ddst_ref[0, :] + ddst_ref[1, :], 1.0))
    h = jnp.maximum((p_ref[0] + p_ref[1]) * nd[:, None] + b_ref[...], 0.0)
    ns = lax.rsqrt(jnp.maximum(dsrc_ref[0, :] + dsrc_ref[1, :], 1.0))
    y_ref[...] = jnp.dot(h, w_ref[...],
                         preferred_element_type=jnp.float32) * ns[:, None]


_comb_mm = pl.pallas_call(
    _comb_mm_body,
    grid=(GRID_TC,),
    in_specs=[
        pl.BlockSpec((NCORE, ROWS_TC, D), lambda i: (0, i, 0)),
        pl.BlockSpec((NCORE, ROWS_TC), lambda i: (0, i)),
        pl.BlockSpec((1, D), lambda i: (0, 0)),
        pl.BlockSpec((NCORE, ROWS_TC), lambda i: (0, i)),
        pl.BlockSpec((D, D), lambda i: (0, 0)),
    ],
    out_specs=pl.BlockSpec((ROWS_TC, D), lambda i: (i, 0)),
    out_shape=jax.ShapeDtypeStruct((NP, D), jnp.float32),
)


def _final_body(p_ref, ddst_ref, b_ref, out_ref):
    nd = lax.rsqrt(jnp.maximum(ddst_ref[0, :] + ddst_ref[1, :], 1.0))
    z = jnp.maximum((p_ref[0] + p_ref[1]) * nd[:, None] + b_ref[...], 0.0)
    z = z - jnp.max(z, axis=1, keepdims=True)
    e = jnp.exp(z)
    out_ref[...] = e / jnp.sum(e, axis=1, keepdims=True)


_final = pl.pallas_call(
    _final_body,
    grid=(GRID_TC,),
    in_specs=[
        pl.BlockSpec((NCORE, ROWS_TC, D), lambda i: (0, i, 0)),
        pl.BlockSpec((NCORE, ROWS_TC), lambda i: (0, i)),
        pl.BlockSpec((1, D), lambda i: (0, 0)),
    ],
    out_specs=pl.BlockSpec((ROWS_TC, D), lambda i: (i, 0)),
    out_shape=jax.ShapeDtypeStruct((N_NODES, D), jnp.float32),
)


def kernel(in_feat, edge_index, W1, b1, W2, b2):
    e2 = edge_index.reshape(2 * NBT, B_EDGE)
    ones = jnp.ones((B_EDGE,), jnp.float32)
    z1 = jnp.zeros((RPW,), jnp.float32)
    z2 = jnp.zeros((ZR, D), jnp.float32)
    degp = _deg_kernel(e2, ones, z1)
    dsrc = degp[:, 0, :]
    ddst = degp[:, 1, :]
    b1r = b1.reshape(1, D)
    b2r = b2.reshape(1, D)
    y1 = _mm_scale(in_feat, W1, dsrc)
    p1 = _agg_kernel(y1, e2, z2)
    y2 = _comb_mm(p1, ddst, b1r, dsrc, W2)
    p2 = _agg_kernel(y2, e2, z2)
    return _final(p2, ddst, b2r)


# speedup vs baseline: 1.3722x; 1.0121x over previous
import functools

import jax
import jax.numpy as jnp
from jax import lax
from jax.experimental import pallas as pl
from jax.experimental.pallas import tpu as pltpu
from jax.experimental.pallas import tpu_sc as plsc

N_NODES = 10000
NP = 10240
D = 128
E_EDGES = 320000
NCORE = 2
NSUB = 16
NW = NCORE * NSUB
EPW = E_EDGES // NW
B_EDGE = 128
NBT = E_EDGES // B_EDGE
NB_FLOOR = NBT // NW
REM = NBT - NB_FLOOR * NW
NB_MAX = NB_FLOOR + 1
RPW = NP // NSUB
ACC_R = 10112
ARPW = ACC_R // NSUB
ZR = 128
ROWS_TC = 1024
GRID_TC = NP // ROWS_TC

_mesh = plsc.VectorSubcoreMesh(core_axis_name="c", subcore_axis_name="s")


def _fill_batch_row_ids(rowids_ref, base, nb):
    for i in range((NB_MAX + 15) // 16 * 16 // 16):
        loc = jnp.minimum(lax.iota(jnp.int32, 16) + (i * 16), nb - 1)
        rowids_ref[pl.ds(i * 16, 16)] = loc + base


@functools.partial(
    pl.kernel,
    out_type=jax.ShapeDtypeStruct((NCORE, 2, NP), jnp.float32),
    mesh=_mesh,
    scratch_types=[
        pltpu.VMEM((80,), jnp.int32),
        pltpu.VMEM((80,), jnp.int32),
        pltpu.VMEM((80, B_EDGE), jnp.int32),
        pltpu.VMEM((80, B_EDGE), jnp.int32),
        pltpu.VMEM((B_EDGE,), jnp.float32),
        pltpu.VMEM_SHARED((NP,), jnp.float32),
        pltpu.VMEM_SHARED((NP,), jnp.float32),
        pltpu.SemaphoreType.DMA,
        pltpu.SemaphoreType.DMA,
    ],
)
def _deg_kernel(e2_hbm, ones_hbm, z1_hbm, out_hbm, rowids, rowids2, src_all,
                dst_all, ones_v, hsrc, hdst, sem0, sem1):
    c = lax.axis_index("c")
    s = lax.axis_index("s")
    wid = c * NSUB + s
    nb = NB_FLOOR + jnp.where(wid < REM, 1, 0)
    rbase = wid * NB_FLOOR + jnp.minimum(wid, REM)
    _fill_batch_row_ids(rowids, rbase, nb)
    _fill_batch_row_ids(rowids2, NBT + rbase, nb)
    a0 = pltpu.async_copy(e2_hbm.at[rowids], src_all, sem0)
    a1 = pltpu.async_copy(e2_hbm.at[rowids2], dst_all, sem1)
    pltpu.sync_copy(z1_hbm, hsrc.at[pl.ds(s * RPW, RPW)])
    pltpu.sync_copy(z1_hbm, hdst.at[pl.ds(s * RPW, RPW)])
    pltpu.sync_copy(ones_hbm, ones_v)
    a0.wait()
    a1.wait()
    plsc.subcore_barrier()

    pltpu.async_copy(ones_v, hsrc.at[src_all.at[0]], sem0, add=True)
    pltpu.async_copy(ones_v, hdst.at[dst_all.at[0]], sem1, add=True)

    def body(j, carry):
        pltpu.async_copy(ones_v, hsrc.at[src_all.at[j]], sem0, add=True)
        pltpu.async_copy(ones_v, hdst.at[dst_all.at[j]], sem1, add=True)
        pltpu.make_async_copy(ones_hbm, ones_v, sem0).wait()
        pltpu.make_async_copy(ones_hbm, ones_v, sem1).wait()
        return carry

    lax.fori_loop(1, nb, body, 0)
    pltpu.make_async_copy(ones_hbm, ones_v, sem0).wait()
    pltpu.make_async_copy(ones_hbm, ones_v, sem1).wait()
    plsc.subcore_barrier()
    pltpu.sync_copy(hsrc.at[pl.ds(s * RPW, RPW)], out_hbm.at[c, 0, pl.ds(s * RPW, RPW)])
    pltpu.sync_copy(hdst.at[pl.ds(s * RPW, RPW)], out_hbm.at[c, 1, pl.ds(s * RPW, RPW)])


@functools.partial(
    pl.kernel,
    out_type=jax.ShapeDtypeStruct((NCORE, ACC_R, D), jnp.float32),
    mesh=_mesh,
    scratch_types=[
        pltpu.VMEM((80,), jnp.int32),
        pltpu.VMEM((80, B_EDGE), jnp.int32),
        pltpu.VMEM((B_EDGE,), jnp.int32),
        pltpu.VMEM((B_EDGE,), jnp.int32),
        pltpu.VMEM((B_EDGE, D), jnp.float32),
        pltpu.VMEM((B_EDGE, D), jnp.float32),
        pltpu.VMEM_SHARED((ACC_R, D), jnp.float32),
        pltpu.SemaphoreType.DMA,
        pltpu.SemaphoreType.DMA,
        pltpu.SemaphoreType.DMA,
        pltpu.SemaphoreType.DMA,
        pltpu.SemaphoreType.DMA,
    ],
)
def _agg_kernel(y_hbm, e2_hbm, z2_hbm, out_hbm, rowids, src_all,
                dst_v0, dst_v1, rows0, rows1, acc, semi, semd0, semd1,
                sem0, sem1):
    c = lax.axis_index("c")
    s = lax.axis_index("s")
    wid = c * NSUB + s
    nb = NB_FLOOR + jnp.where(wid < REM, 1, 0)
    rbase = wid * NB_FLOOR + jnp.minimum(wid, REM)
    dbase = NBT + rbase
    _fill_batch_row_ids(rowids, rbase, nb)
    a0 = pltpu.async_copy(e2_hbm.at[rowids], src_all, semi)
    pltpu.async_copy(e2_hbm.at[dbase], dst_v0, semd0)
    pltpu.async_copy(e2_hbm.at[dbase + 1], dst_v1, semd1)
    for k in range(ARPW // ZR):
        pltpu.sync_copy(z2_hbm, acc.at[pl.ds(s * ARPW + k * ZR, ZR)])
    pltpu.sync_copy(z2_hbm.at[pl.ds(0, ARPW % ZR)],
                    acc.at[pl.ds(s * ARPW + (ARPW // ZR) * ZR, ARPW % ZR)])
    a0.wait()
    pltpu.async_copy(y_hbm.at[src_all.at[0]], rows0, sem0)
    pltpu.async_copy(y_hbm.at[src_all.at[1]], rows1, sem1)
    plsc.subcore_barrier()

    def body(jj, carry):
        j = jj * 2
        jn0 = jnp.minimum(j + 2, nb - 1)
        jn1 = jnp.minimum(j + 3, nb - 1)
        pltpu.make_async_copy(z2_hbm, rows0, sem0).wait()
        pltpu.make_async_copy(e2_hbm.at[dbase], dst_v0, semd0).wait()
        pltpu.sync_copy(rows0, acc.at[dst_v0], add=True)
        pltpu.async_copy(e2_hbm.at[dbase + jn0], dst_v0, semd0)
        pltpu.async_copy(y_hbm.at[src_all.at[jn0]], rows0, sem0)
        pltpu.make_async_copy(z2_hbm, rows1, sem1).wait()
        pltpu.make_async_copy(e2_hbm.at[dbase], dst_v1, semd1).wait()
        pltpu.sync_copy(rows1, acc.at[dst_v1], add=True)
        pltpu.async_copy(e2_hbm.at[dbase + jn1], dst_v1, semd1)
        pltpu.async_copy(y_hbm.at[src_all.at[jn1]], rows1, sem1)
        return carry

    lax.fori_loop(0, NB_FLOOR // 2, body, 0)
    pltpu.make_async_copy(z2_hbm, rows0, sem0).wait()
    pltpu.make_async_copy(e2_hbm.at[dbase], dst_v0, semd0).wait()

    @pl.when(wid < REM)
    def _():
        pltpu.sync_copy(rows0, acc.at[dst_v0], add=True)

    pltpu.make_async_copy(z2_hbm, rows1, sem1).wait()
    pltpu.make_async_copy(e2_hbm.at[dbase], dst_v1, semd1).wait()
    plsc.subcore_barrier()
    pltpu.sync_copy(acc.at[pl.ds(s * ARPW, ARPW)], out_hbm.at[c, pl.ds(s * ARPW, ARPW)])


def _mm_scale_body(x_ref, w_ref, dsrc_ref, y_ref):
    deg = dsrc_ref[0, :] + dsrc_ref[1, :]
    ns = lax.rsqrt(jnp.maximum(deg, 1.0))
    y_ref[...] = jnp.dot(x_ref[...], w_ref[...],
                         preferred_element_type=jnp.float32) * ns[:, None]


_mm_scale = pl.pallas_call(
    _mm_scale_body,
    grid=(GRID_TC,),
    in_specs=[
        pl.BlockSpec((ROWS_TC, D), lambda i: (i, 0)),
        pl.BlockSpec((D, D), lambda i: (0, 0)),
        pl.BlockSpec((NCORE, ROWS_TC), lambda i: (0, i)),
    ],
    out_specs=pl.BlockSpec((ROWS_TC, D), lambda i: (i, 0)),
    out_shape=jax.ShapeDtypeStruct((NP, D), jnp.float32),
)


def _comb_mm_body(p_ref, ddst_ref, b_ref, dsrc_ref, w_ref, y_ref):
    nd = lax.rsqrt(jnp.maximum(ddst_ref[0, :] + ddst_ref[1, :], 1.0))
    h = jnp.maximum((p_ref[0] + p_ref[1]) * nd[:, None] + b_ref[...], 0.0)
    ns = lax.rsqrt(jnp.maximum(dsrc_ref[0, :] + dsrc_ref[1, :], 1.0))
    y_ref[...] = jnp.dot(h, w_ref[...],
                         preferred_element_type=jnp.float32) * ns[:, None]


_comb_mm = pl.pallas_call(
    _comb_mm_body,
    grid=(GRID_TC,),
    in_specs=[
        pl.BlockSpec((NCORE, ROWS_TC, D), lambda i: (0, i, 0)),
        pl.BlockSpec((NCORE, ROWS_TC), lambda i: (0, i)),
        pl.BlockSpec((1, D), lambda i: (0, 0)),
        pl.BlockSpec((NCORE, ROWS_TC), lambda i: (0, i)),
        pl.BlockSpec((D, D), lambda i: (0, 0)),
    ],
    out_specs=pl.BlockSpec((ROWS_TC, D), lambda i: (i, 0)),
    out_shape=jax.ShapeDtypeStruct((NP, D), jnp.float32),
)


def _final_body(p_ref, ddst_ref, b_ref, out_ref):
    nd = lax.rsqrt(jnp.maximum(ddst_ref[0, :] + ddst_ref[1, :], 1.0))
    z = jnp.maximum((p_ref[0] + p_ref[1]) * nd[:, None] + b_ref[...], 0.0)
    z = z - jnp.max(z, axis=1, keepdims=True)
    e = jnp.exp(z)
    out_ref[...] = e / jnp.sum(e, axis=1, keepdims=True)


_final = pl.pallas_call(
    _final_body,
    grid=(GRID_TC,),
    in_specs=[
        pl.BlockSpec((NCORE, ROWS_TC, D), lambda i: (0, i, 0)),
        pl.BlockSpec((NCORE, ROWS_TC), lambda i: (0, i)),
        pl.BlockSpec((1, D), lambda i: (0, 0)),
    ],
    out_specs=pl.BlockSpec((ROWS_TC, D), lambda i: (i, 0)),
    out_shape=jax.ShapeDtypeStruct((N_NODES, D), jnp.float32),
)


def kernel(in_feat, edge_index, W1, b1, W2, b2):
    e2 = edge_index.reshape(2 * NBT, B_EDGE)
    ones = jnp.ones((B_EDGE,), jnp.float32)
    z1 = jnp.zeros((RPW,), jnp.float32)
    z2 = jnp.zeros((ZR, D), jnp.float32)
    degp = _deg_kernel(e2, ones, z1)
    dsrc = degp[:, 0, :]
    ddst = degp[:, 1, :]
    b1r = b1.reshape(1, D)
    b2r = b2.reshape(1, D)
    y1 = _mm_scale(in_feat, W1, dsrc)
    p1 = _agg_kernel(y1, e2, z2)
    y2 = _comb_mm(p1, ddst, b1r, dsrc, W2)
    p2 = _agg_kernel(y2, e2, z2)
    return _final(p2, ddst, b2r)


# speedup vs baseline: 1.3761x; 1.0028x over previous
import functools

import jax
import jax.numpy as jnp
from jax import lax
from jax.experimental import pallas as pl
from jax.experimental.pallas import tpu as pltpu
from jax.experimental.pallas import tpu_sc as plsc

N_NODES = 10000
NP = 10240
D = 128
E_EDGES = 320000
NCORE = 2
NSUB = 16
NW = NCORE * NSUB
B_EDGE = 128
NBT = E_EDGES // B_EDGE
NB_FLOOR = NBT // NW
REM = NBT - NB_FLOOR * NW
NB_MAX = NB_FLOOR + 1
RPW = NP // NSUB
ACC_R = 10112
ARPW = ACC_R // NSUB
ZR = 128
ROWS_TC = 1024
GRID_TC = NP // ROWS_TC

_mesh = plsc.VectorSubcoreMesh(core_axis_name="c", subcore_axis_name="s")


def _fill_batch_row_ids(rowids_ref, base, nb):
    for i in range((NB_MAX + 15) // 16 * 16 // 16):
        loc = jnp.minimum(lax.iota(jnp.int32, 16) + (i * 16), nb - 1)
        rowids_ref[pl.ds(i * 16, 16)] = loc + base


@functools.partial(
    pl.kernel,
    out_type=jax.ShapeDtypeStruct((NCORE, 2, NP), jnp.float32),
    mesh=_mesh,
    scratch_types=[
        pltpu.VMEM((80,), jnp.int32),
        pltpu.VMEM((80,), jnp.int32),
        pltpu.VMEM((80, B_EDGE), jnp.int32),
        pltpu.VMEM((80, B_EDGE), jnp.int32),
        pltpu.VMEM((B_EDGE,), jnp.float32),
        pltpu.VMEM_SHARED((NP,), jnp.float32),
        pltpu.VMEM_SHARED((NP,), jnp.float32),
        pltpu.SemaphoreType.DMA,
        pltpu.SemaphoreType.DMA,
    ],
)
def _deg_kernel(e2_hbm, ones_hbm, z1_hbm, out_hbm, rowids, rowids2, src_all,
                dst_all, ones_v, hsrc, hdst, sem0, sem1):
    c = lax.axis_index("c")
    s = lax.axis_index("s")
    wid = c * NSUB + s
    nb = NB_FLOOR + jnp.where(wid < REM, 1, 0)
    rbase = wid * NB_FLOOR + jnp.minimum(wid, REM)
    _fill_batch_row_ids(rowids, rbase, nb)
    _fill_batch_row_ids(rowids2, NBT + rbase, nb)
    a0 = pltpu.async_copy(e2_hbm.at[rowids], src_all, sem0)
    a1 = pltpu.async_copy(e2_hbm.at[rowids2], dst_all, sem1)
    pltpu.sync_copy(z1_hbm, hsrc.at[pl.ds(s * RPW, RPW)])
    pltpu.sync_copy(z1_hbm, hdst.at[pl.ds(s * RPW, RPW)])
    pltpu.sync_copy(ones_hbm, ones_v)
    a0.wait()
    a1.wait()
    plsc.subcore_barrier()

    pltpu.async_copy(ones_v, hsrc.at[src_all.at[0]], sem0, add=True)
    pltpu.async_copy(ones_v, hdst.at[dst_all.at[0]], sem1, add=True)

    def body(j, carry):
        pltpu.async_copy(ones_v, hsrc.at[src_all.at[j]], sem0, add=True)
        pltpu.async_copy(ones_v, hdst.at[dst_all.at[j]], sem1, add=True)
        pltpu.make_async_copy(ones_hbm, ones_v, sem0).wait()
        pltpu.make_async_copy(ones_hbm, ones_v, sem1).wait()
        return carry

    lax.fori_loop(1, nb, body, 0)
    pltpu.make_async_copy(ones_hbm, ones_v, sem0).wait()
    pltpu.make_async_copy(ones_hbm, ones_v, sem1).wait()
    plsc.subcore_barrier()
    pltpu.sync_copy(hsrc.at[pl.ds(s * RPW, RPW)], out_hbm.at[c, 0, pl.ds(s * RPW, RPW)])
    pltpu.sync_copy(hdst.at[pl.ds(s * RPW, RPW)], out_hbm.at[c, 1, pl.ds(s * RPW, RPW)])


@functools.partial(
    pl.kernel,
    out_type=jax.ShapeDtypeStruct((NCORE, ACC_R, D), jnp.float32),
    mesh=_mesh,
    scratch_types=[
        pltpu.VMEM((80,), jnp.int32),
        pltpu.VMEM((80, B_EDGE), jnp.int32),
        pltpu.VMEM((B_EDGE,), jnp.int32),
        pltpu.VMEM((B_EDGE,), jnp.int32),
        pltpu.VMEM((B_EDGE, D), jnp.float32),
        pltpu.VMEM((B_EDGE, D), jnp.float32),
        pltpu.VMEM_SHARED((ACC_R, D), jnp.float32),
        pltpu.SemaphoreType.DMA,
        pltpu.SemaphoreType.DMA,
        pltpu.SemaphoreType.DMA,
        pltpu.SemaphoreType.DMA,
        pltpu.SemaphoreType.DMA,
    ],
)
def _agg_kernel(y_hbm, e2_hbm, z2_hbm, out_hbm, rowids, src_all,
                dst_v0, dst_v1, rows0, rows1, acc, semi, semd0, semd1,
                sem0, sem1):
    c = lax.axis_index("c")
    s = lax.axis_index("s")
    wid = c * NSUB + s
    nb = NB_FLOOR + jnp.where(wid < REM, 1, 0)
    rbase = wid * NB_FLOOR + jnp.minimum(wid, REM)
    dbase = NBT + rbase
    _fill_batch_row_ids(rowids, rbase, nb)
    a0 = pltpu.async_copy(e2_hbm.at[rowids], src_all, semi)
    pltpu.async_copy(e2_hbm.at[dbase], dst_v0, semd0)
    pltpu.async_copy(e2_hbm.at[dbase + 1], dst_v1, semd1)
    for k in range(ARPW // ZR):
        pltpu.sync_copy(z2_hbm, acc.at[pl.ds(s * ARPW + k * ZR, ZR)])
    pltpu.sync_copy(z2_hbm.at[pl.ds(0, ARPW % ZR)],
                    acc.at[pl.ds(s * ARPW + (ARPW // ZR) * ZR, ARPW % ZR)])
    a0.wait()
    pltpu.async_copy(y_hbm.at[src_all.at[0]], rows0, sem0)
    pltpu.async_copy(y_hbm.at[src_all.at[1]], rows1, sem1)
    plsc.subcore_barrier()

    def body(jj, carry):
        j = jj * 2
        jn0 = jnp.minimum(j + 2, nb - 1)
        jn1 = jnp.minimum(j + 3, nb - 1)
        pltpu.make_async_copy(z2_hbm, rows0, sem0).wait()
        pltpu.make_async_copy(e2_hbm.at[dbase], dst_v0, semd0).wait()
        pltpu.sync_copy(rows0, acc.at[dst_v0], add=True)
        pltpu.async_copy(e2_hbm.at[dbase + jn0], dst_v0, semd0)
        pltpu.async_copy(y_hbm.at[src_all.at[jn0]], rows0, sem0)
        pltpu.make_async_copy(z2_hbm, rows1, sem1).wait()
        pltpu.make_async_copy(e2_hbm.at[dbase], dst_v1, semd1).wait()
        pltpu.sync_copy(rows1, acc.at[dst_v1], add=True)
        pltpu.async_copy(e2_hbm.at[dbase + jn1], dst_v1, semd1)
        pltpu.async_copy(y_hbm.at[src_all.at[jn1]], rows1, sem1)
        return carry

    lax.fori_loop(0, NB_FLOOR // 2, body, 0)
    pltpu.make_async_copy(z2_hbm, rows0, sem0).wait()
    pltpu.make_async_copy(e2_hbm.at[dbase], dst_v0, semd0).wait()

    @pl.when(wid < REM)
    def _():
        pltpu.sync_copy(rows0, acc.at[dst_v0], add=True)

    pltpu.make_async_copy(z2_hbm, rows1, sem1).wait()
    pltpu.make_async_copy(e2_hbm.at[dbase], dst_v1, semd1).wait()
    plsc.subcore_barrier()
    pltpu.sync_copy(acc.at[pl.ds(s * ARPW, ARPW)], out_hbm.at[c, pl.ds(s * ARPW, ARPW)])


def _mm_scale_body(x_ref, w_ref, dsrc_ref, y_ref):
    deg = dsrc_ref[0, :] + dsrc_ref[1, :]
    ns = lax.rsqrt(jnp.maximum(deg, 1.0))
    y_ref[...] = jnp.dot(x_ref[...], w_ref[...],
                         preferred_element_type=jnp.float32) * ns[:, None]


_mm_scale = pl.pallas_call(
    _mm_scale_body,
    grid=(GRID_TC,),
    in_specs=[
        pl.BlockSpec((ROWS_TC, D), lambda i: (i, 0)),
        pl.BlockSpec((D, D), lambda i: (0, 0)),
        pl.BlockSpec((NCORE, ROWS_TC), lambda i: (0, i)),
    ],
    out_specs=pl.BlockSpec((ROWS_TC, D), lambda i: (i, 0)),
    out_shape=jax.ShapeDtypeStruct((NP, D), jnp.float32),
)


def _comb_mm_body(p_ref, ddst_ref, b_ref, dsrc_ref, w_ref, y_ref):
    nd = lax.rsqrt(jnp.maximum(ddst_ref[0, :] + ddst_ref[1, :], 1.0))
    h = jnp.maximum((p_ref[0] + p_ref[1]) * nd[:, None] + b_ref[...], 0.0)
    ns = lax.rsqrt(jnp.maximum(dsrc_ref[0, :] + dsrc_ref[1, :], 1.0))
    y_ref[...] = jnp.dot(h, w_ref[...],
                         preferred_element_type=jnp.float32) * ns[:, None]


_comb_mm = pl.pallas_call(
    _comb_mm_body,
    grid=(GRID_TC,),
    in_specs=[
        pl.BlockSpec((NCORE, ROWS_TC, D), lambda i: (0, i, 0)),
        pl.BlockSpec((NCORE, ROWS_TC), lambda i: (0, i)),
        pl.BlockSpec((1, D), lambda i: (0, 0)),
        pl.BlockSpec((NCORE, ROWS_TC), lambda i: (0, i)),
        pl.BlockSpec((D, D), lambda i: (0, 0)),
    ],
    out_specs=pl.BlockSpec((ROWS_TC, D), lambda i: (i, 0)),
    out_shape=jax.ShapeDtypeStruct((NP, D), jnp.float32),
)


def _final_body(p_ref, ddst_ref, b_ref, out_ref):
    nd = lax.rsqrt(jnp.maximum(ddst_ref[0, :] + ddst_ref[1, :], 1.0))
    z = jnp.maximum((p_ref[0] + p_ref[1]) * nd[:, None] + b_ref[...], 0.0)
    z = z - jnp.max(z, axis=1, keepdims=True)
    e = jnp.exp(z)
    out_ref[...] = e / jnp.sum(e, axis=1, keepdims=True)


_final = pl.pallas_call(
    _final_body,
    grid=(GRID_TC,),
    in_specs=[
        pl.BlockSpec((NCORE, ROWS_TC, D), lambda i: (0, i, 0)),
        pl.BlockSpec((NCORE, ROWS_TC), lambda i: (0, i)),
        pl.BlockSpec((1, D), lambda i: (0, 0)),
    ],
    out_specs=pl.BlockSpec((ROWS_TC, D), lambda i: (i, 0)),
    out_shape=jax.ShapeDtypeStruct((N_NODES, D), jnp.float32),
)


def kernel(in_feat, edge_index, W1, b1, W2, b2):
    e2 = edge_index.reshape(2 * NBT, B_EDGE)
    ones = jnp.ones((B_EDGE,), jnp.float32)
    z1 = jnp.zeros((RPW,), jnp.float32)
    z2 = jnp.zeros((ZR, D), jnp.float32)
    degp = _deg_kernel(e2, ones, z1)
    dsrc = degp[:, 0, :]
    ddst = degp[:, 1, :]
    b1r = b1.reshape(1, D)
    b2r = b2.reshape(1, D)
    y1 = _mm_scale(in_feat, W1, dsrc)
    p1 = _agg_kernel(y1, e2, z2)
    y2 = _comb_mm(p1, ddst, b1r, dsrc, W2)
    p2 = _agg_kernel(y2, e2, z2)
    return _final(p2, ddst, b2r)
